# Initial kernel scaffold; baseline (speedup 1.0000x reference)
import functools

import jax
import jax.numpy as jnp
from jax import lax
from jax.experimental import pallas as pl
from jax.experimental.pallas import tpu as pltpu
from jax.experimental.pallas import tpu_sc as plsc

N = 10000
NPAD = 10016
E = 320000
NW = 32
NB = 80
BATCH = 128
EPAD = NW * NB * BATCH
RPT = NPAD // 16

_mesh = plsc.VectorSubcoreMesh(core_axis_name="c", subcore_axis_name="s")


def _seg_body(p_hbm, src_hbm, dst_hbm, zrow_hbm, z16_hbm, ones_hbm,
              sum_out, cnt_out, idx_s, idx_d, rows, ones_v, acc, cnt, sem,
              *, with_cnt):
    cid = lax.axis_index("c")
    sid = lax.axis_index("s")
    wid = cid * 16 + sid
    r0 = sid * RPT
    pltpu.sync_copy(zrow_hbm, acc.at[pl.ds(r0, RPT)])
    if with_cnt:
        pltpu.sync_copy(z16_hbm, cnt.at[pl.ds(r0, RPT)])
        pltpu.sync_copy(ones_hbm, ones_v)
    pltpu.sync_copy(src_hbm.at[wid], idx_s)
    pltpu.sync_copy(dst_hbm.at[wid], idx_d)
    plsc.subcore_barrier()

    def body(j, carry):
        pltpu.async_copy(p_hbm.at[idx_s.at[j]], rows, sem).wait()
        pltpu.sync_copy(rows, acc.at[idx_d.at[j]], add=True)
        if with_cnt:
            pltpu.sync_copy(ones_v, cnt.at[idx_d.at[j]], add=True)
        return carry

    lax.fori_loop(0, NB, body, 0)
    plsc.subcore_barrier()
    pltpu.sync_copy(acc.at[pl.ds(r0, RPT)], sum_out.at[cid, pl.ds(r0, RPT)])
    if with_cnt:
        pltpu.sync_copy(cnt.at[pl.ds(r0, RPT)], cnt_out.at[cid, pl.ds(r0, RPT)])


def _make_sc_layer(width, with_cnt):
    out_type = [jax.ShapeDtypeStruct((2, NPAD, width), jnp.float32)]
    if with_cnt:
        out_type.append(jax.ShapeDtypeStruct((2, NPAD, 16), jnp.float32))
    scratch = [
        pltpu.VMEM((NB, BATCH), jnp.int32),
        pltpu.VMEM((NB, BATCH), jnp.int32),
        pltpu.VMEM((BATCH, width), jnp.float32),
        pltpu.VMEM((BATCH, 16), jnp.float32),
        pltpu.VMEM_SHARED((NPAD, width), jnp.float32),
        pltpu.VMEM_SHARED((NPAD, 16), jnp.float32),
        pltpu.SemaphoreType.DMA,
    ]
    body = functools.partial(_seg_body, with_cnt=with_cnt)
    return pl.kernel(body, out_type=out_type, mesh=_mesh,
                     scratch_types=scratch)


_sc_layer1 = _make_sc_layer(32, True)
_sc_layer2 = _make_sc_layer(16, False)


def _tc_in_body(x_ref, wl_ref, wr_ref, bl_ref, p_ref, q_ref):
    x = x_ref[...]
    p_ref[...] = jnp.dot(x, wl_ref[...], preferred_element_type=jnp.float32)
    q_ref[...] = (jnp.dot(x, wr_ref[...], preferred_element_type=jnp.float32)
                  + bl_ref[...])


def _tc_mid_body(sum_ref, cnt_ref, q1_ref, wl_ref, wr_ref, bl_ref,
                 p2_ref, q2_ref):
    s = sum_ref[0, :N, :] + sum_ref[1, :N, :]
    c = cnt_ref[0, :N, :1] + cnt_ref[1, :N, :1]
    h = jnp.maximum(s / jnp.maximum(c, 1.0) + q1_ref[...], 0.0)
    p2_ref[...] = jnp.dot(h, wl_ref[...], preferred_element_type=jnp.float32)
    q2_ref[...] = (jnp.dot(h, wr_ref[...], preferred_element_type=jnp.float32)
                   + bl_ref[...])


def _tc_out_body(sum_ref, cnt_ref, q2_ref, wo_ref, bo_ref, o_ref):
    s = sum_ref[0, :N, :] + sum_ref[1, :N, :]
    c = cnt_ref[0, :N, :1] + cnt_ref[1, :N, :1]
    h = jnp.maximum(s / jnp.maximum(c, 1.0) + q2_ref[...], 0.0)
    o_ref[...] = jax.nn.sigmoid(
        jnp.dot(h, wo_ref[...], preferred_element_type=jnp.float32)
        + bo_ref[...])


_tc_in = pl.pallas_call(
    _tc_in_body,
    out_shape=[jax.ShapeDtypeStruct((N, 32), jnp.float32),
               jax.ShapeDtypeStruct((N, 32), jnp.float32)])
_tc_mid = pl.pallas_call(
    _tc_mid_body,
    out_shape=[jax.ShapeDtypeStruct((N, 16), jnp.float32),
               jax.ShapeDtypeStruct((N, 16), jnp.float32)])
_tc_out = pl.pallas_call(
    _tc_out_body,
    out_shape=jax.ShapeDtypeStruct((N, 8), jnp.float32))


def kernel(x, edge_index, Wl1, bl1, Wr1, Wl2, bl2, Wr2, Wo, bo):
    ei = edge_index.astype(jnp.int32)
    src = jnp.concatenate(
        [ei[0], jnp.zeros((EPAD - E,), jnp.int32)]).reshape(NW, NB, BATCH)
    dst = jnp.concatenate(
        [ei[1], jnp.full((EPAD - E,), N, jnp.int32)]).reshape(NW, NB, BATCH)
    z32 = jnp.zeros((RPT, 32), jnp.float32)
    z16 = jnp.zeros((RPT, 16), jnp.float32)
    ones = jnp.ones((BATCH, 16), jnp.float32)

    p1, q1 = _tc_in(x, Wl1, Wr1, bl1.reshape(1, 32))
    sum1, cnt = _sc_layer1(p1, src, dst, z32, z16, ones)
    p2, q2 = _tc_mid(sum1, cnt, q1, Wl2, Wr2, bl2.reshape(1, 16))
    sum2, = _sc_layer2(p2, src, dst, z16, z16, ones)
    o = _tc_out(sum2, cnt, q2, jnp.tile(Wo, (1, 8)), bo.reshape(1, 1))
    return o[:, :1]


# speedup vs baseline: 9.5065x; 9.5065x over previous
import functools

import jax
import jax.numpy as jnp
from jax import lax
from jax.experimental import pallas as pl
from jax.experimental.pallas import tpu as pltpu
from jax.experimental.pallas import tpu_sc as plsc

N = 10000
NPAD = 10112
E = 320000
NW = 32
NB = 80
BATCH = 128
EPAD = NW * NB * BATCH
RPT = NPAD // 16

_mesh = plsc.VectorSubcoreMesh(core_axis_name="c", subcore_axis_name="s")


def _sc_layer1_body(p_hbm, src_hbm, dst_hbm, z32_hbm, z16_hbm, ones_hbm,
                    sum_out, cnt_out, idx_s, idx_d, rows, ones_v, acc, cnt,
                    sem):
    cid = lax.axis_index("c")
    sid = lax.axis_index("s")
    wid = cid * 16 + sid
    r0 = sid * RPT
    pltpu.sync_copy(z32_hbm, acc.at[pl.ds(r0, RPT)])
    pltpu.sync_copy(z16_hbm, cnt.at[pl.ds(r0, RPT)])
    pltpu.sync_copy(ones_hbm, ones_v)
    pltpu.sync_copy(src_hbm.at[wid], idx_s)
    pltpu.sync_copy(dst_hbm.at[wid], idx_d)
    plsc.subcore_barrier()

    def body(j, carry):
        pltpu.async_copy(p_hbm.at[idx_s.at[j]], rows, sem).wait()
        pltpu.sync_copy(rows, acc.at[idx_d.at[j]], add=True)
        pltpu.sync_copy(ones_v, cnt.at[idx_d.at[j]], add=True)
        return carry

    lax.fori_loop(0, NB, body, 0)
    plsc.subcore_barrier()
    pltpu.sync_copy(acc.at[pl.ds(r0, RPT)], sum_out.at[cid, pl.ds(r0, RPT)])
    pltpu.sync_copy(cnt.at[pl.ds(r0, RPT)], cnt_out.at[cid, pl.ds(r0, RPT)])


def _sc_layer2_body(p_hbm, src_hbm, dst_hbm, z16_hbm,
                    sum_out, idx_s, idx_d, rows, acc, sem):
    cid = lax.axis_index("c")
    sid = lax.axis_index("s")
    wid = cid * 16 + sid
    r0 = sid * RPT
    pltpu.sync_copy(z16_hbm, acc.at[pl.ds(r0, RPT)])
    pltpu.sync_copy(src_hbm.at[wid], idx_s)
    pltpu.sync_copy(dst_hbm.at[wid], idx_d)
    plsc.subcore_barrier()

    def body(j, carry):
        pltpu.async_copy(p_hbm.at[idx_s.at[j]], rows, sem).wait()
        pltpu.sync_copy(rows, acc.at[idx_d.at[j]], add=True)
        return carry

    lax.fori_loop(0, NB, body, 0)
    plsc.subcore_barrier()
    pltpu.sync_copy(acc.at[pl.ds(r0, RPT)], sum_out.at[cid, pl.ds(r0, RPT)])


_sc_layer1 = pl.kernel(
    _sc_layer1_body,
    out_type=[jax.ShapeDtypeStruct((2, NPAD, 32), jnp.float32),
              jax.ShapeDtypeStruct((2, NPAD, 16), jnp.float32)],
    mesh=_mesh,
    compiler_params=pltpu.CompilerParams(use_tc_tiling_on_sc=False),
    scratch_types=[
        pltpu.VMEM((NB, BATCH), jnp.int32),
        pltpu.VMEM((NB, BATCH), jnp.int32),
        pltpu.VMEM((BATCH, 32), jnp.float32),
        pltpu.VMEM((BATCH, 16), jnp.float32),
        pltpu.VMEM_SHARED((NPAD, 32), jnp.float32),
        pltpu.VMEM_SHARED((NPAD, 16), jnp.float32),
        pltpu.SemaphoreType.DMA,
    ])

_sc_layer2 = pl.kernel(
    _sc_layer2_body,
    out_type=[jax.ShapeDtypeStruct((2, NPAD, 16), jnp.float32)],
    mesh=_mesh,
    compiler_params=pltpu.CompilerParams(use_tc_tiling_on_sc=False),
    scratch_types=[
        pltpu.VMEM((NB, BATCH), jnp.int32),
        pltpu.VMEM((NB, BATCH), jnp.int32),
        pltpu.VMEM((BATCH, 16), jnp.float32),
        pltpu.VMEM_SHARED((NPAD, 16), jnp.float32),
        pltpu.SemaphoreType.DMA,
    ])


def _tc_in_body(x_ref, wl_ref, wr_ref, bl_ref, p_ref, q_ref):
    x = x_ref[...]
    p_ref[...] = jnp.dot(x, wl_ref[...], preferred_element_type=jnp.float32)
    q_ref[...] = (jnp.dot(x, wr_ref[...], preferred_element_type=jnp.float32)
                  + bl_ref[...])


def _tc_mid_body(sum_ref, cnt_ref, q1_ref, wl_ref, wr_ref, bl_ref,
                 p2_ref, q2_ref):
    s = sum_ref[0, :N, :] + sum_ref[1, :N, :]
    c = cnt_ref[0, :N, :1] + cnt_ref[1, :N, :1]
    h = jnp.maximum(s / jnp.maximum(c, 1.0) + q1_ref[...], 0.0)
    p2_ref[...] = jnp.dot(h, wl_ref[...], preferred_element_type=jnp.float32)
    q2_ref[...] = (jnp.dot(h, wr_ref[...], preferred_element_type=jnp.float32)
                   + bl_ref[...])


def _tc_out_body(sum_ref, cnt_ref, q2_ref, wo_ref, bo_ref, o_ref):
    s = sum_ref[0, :N, :] + sum_ref[1, :N, :]
    c = cnt_ref[0, :N, :1] + cnt_ref[1, :N, :1]
    h = jnp.maximum(s / jnp.maximum(c, 1.0) + q2_ref[...], 0.0)
    o_ref[...] = jax.nn.sigmoid(
        jnp.dot(h, wo_ref[...], preferred_element_type=jnp.float32)
        + bo_ref[...])


_tc_in = pl.pallas_call(
    _tc_in_body,
    out_shape=[jax.ShapeDtypeStruct((N, 32), jnp.float32),
               jax.ShapeDtypeStruct((N, 32), jnp.float32)])
_tc_mid = pl.pallas_call(
    _tc_mid_body,
    out_shape=[jax.ShapeDtypeStruct((N, 16), jnp.float32),
               jax.ShapeDtypeStruct((N, 16), jnp.float32)])
_tc_out = pl.pallas_call(
    _tc_out_body,
    out_shape=jax.ShapeDtypeStruct((N, 8), jnp.float32))


def kernel(x, edge_index, Wl1, bl1, Wr1, Wl2, bl2, Wr2, Wo, bo):
    ei = edge_index.astype(jnp.int32)
    src = jnp.concatenate(
        [ei[0], jnp.zeros((EPAD - E,), jnp.int32)]).reshape(NW, NB, BATCH)
    dst = jnp.concatenate(
        [ei[1], jnp.full((EPAD - E,), N, jnp.int32)]).reshape(NW, NB, BATCH)
    z32 = jnp.zeros((RPT, 32), jnp.float32)
    z16 = jnp.zeros((RPT, 16), jnp.float32)
    ones = jnp.ones((BATCH, 16), jnp.float32)

    p1, q1 = _tc_in(x, Wl1, Wr1, bl1.reshape(1, 32))
    sum1, cnt = _sc_layer1(p1, src, dst, z32, z16, ones)
    p2, q2 = _tc_mid(sum1, cnt, q1, Wl2, Wr2, bl2.reshape(1, 16))
    sum2, = _sc_layer2(p2, src, dst, z16)
    o = _tc_out(sum2, cnt, q2, jnp.tile(Wo, (1, 8)), bo.reshape(1, 1))
    return o[:, :1]


# speedup vs baseline: 12.5177x; 1.3167x over previous
import functools

import jax
import jax.numpy as jnp
from jax import lax
from jax.experimental import pallas as pl
from jax.experimental.pallas import tpu as pltpu
from jax.experimental.pallas import tpu_sc as plsc

N = 10000
NPAD = 10112
E = 320000
NW = 32
NB = 80
BATCH = 128
EPAD = NW * NB * BATCH
RPT = NPAD // 16

_mesh = plsc.VectorSubcoreMesh(core_axis_name="c", subcore_axis_name="s")


NG = NB // 2


def _sc_layer1_body(p_hbm, src_hbm, dst_hbm, z32_hbm, z16_hbm, ones_hbm,
                    sum_out, cnt_out, idx_s, idx_d, rows0, rows1, ones_v,
                    acc, cnt, semg0, semg1, semo):
    cid = lax.axis_index("c")
    sid = lax.axis_index("s")
    wid = cid * 16 + sid
    r0 = sid * RPT
    pltpu.sync_copy(z32_hbm, acc.at[pl.ds(r0, RPT)])
    pltpu.sync_copy(z16_hbm, cnt.at[pl.ds(r0, RPT)])
    pltpu.sync_copy(ones_hbm, ones_v)
    pltpu.sync_copy(src_hbm.at[wid], idx_s)
    pltpu.sync_copy(dst_hbm.at[wid], idx_d)
    plsc.subcore_barrier()

    pltpu.async_copy(p_hbm.at[idx_s.at[0]], rows0, semg0)

    def body(g, carry):
        j0 = g * 2
        pltpu.async_copy(p_hbm.at[idx_s.at[j0 + 1]], rows1, semg1)
        pltpu.make_async_copy(p_hbm.at[idx_s.at[0]], rows0, semg0).wait()
        pltpu.sync_copy(rows0, acc.at[idx_d.at[j0]], add=True)
        pltpu.async_copy(ones_v, cnt.at[idx_d.at[j0]], semo, add=True)

        @pl.when(g + 1 < NG)
        def _():
            pltpu.async_copy(p_hbm.at[idx_s.at[j0 + 2]], rows0, semg0)

        pltpu.make_async_copy(p_hbm.at[idx_s.at[0]], rows1, semg1).wait()
        pltpu.sync_copy(rows1, acc.at[idx_d.at[j0 + 1]], add=True)
        pltpu.async_copy(ones_v, cnt.at[idx_d.at[j0 + 1]], semo, add=True)
        return carry

    lax.fori_loop(0, NG, body, 0)

    def drain(j, carry):
        pltpu.make_async_copy(ones_v, cnt.at[idx_d.at[0]], semo).wait()
        return carry

    lax.fori_loop(0, NB, drain, 0)
    plsc.subcore_barrier()
    pltpu.sync_copy(acc.at[pl.ds(r0, RPT)], sum_out.at[cid, pl.ds(r0, RPT)])
    pltpu.sync_copy(cnt.at[pl.ds(r0, RPT)], cnt_out.at[cid, pl.ds(r0, RPT)])


def _sc_layer2_body(p_hbm, src_hbm, dst_hbm, z16_hbm,
                    sum_out, idx_s, idx_d, rows0, rows1, acc, semg0, semg1):
    cid = lax.axis_index("c")
    sid = lax.axis_index("s")
    wid = cid * 16 + sid
    r0 = sid * RPT
    pltpu.sync_copy(z16_hbm, acc.at[pl.ds(r0, RPT)])
    pltpu.sync_copy(src_hbm.at[wid], idx_s)
    pltpu.sync_copy(dst_hbm.at[wid], idx_d)
    plsc.subcore_barrier()

    pltpu.async_copy(p_hbm.at[idx_s.at[0]], rows0, semg0)

    def body(g, carry):
        j0 = g * 2
        pltpu.async_copy(p_hbm.at[idx_s.at[j0 + 1]], rows1, semg1)
        pltpu.make_async_copy(p_hbm.at[idx_s.at[0]], rows0, semg0).wait()
        pltpu.sync_copy(rows0, acc.at[idx_d.at[j0]], add=True)

        @pl.when(g + 1 < NG)
        def _():
            pltpu.async_copy(p_hbm.at[idx_s.at[j0 + 2]], rows0, semg0)

        pltpu.make_async_copy(p_hbm.at[idx_s.at[0]], rows1, semg1).wait()
        pltpu.sync_copy(rows1, acc.at[idx_d.at[j0 + 1]], add=True)
        return carry

    lax.fori_loop(0, NG, body, 0)
    plsc.subcore_barrier()
    pltpu.sync_copy(acc.at[pl.ds(r0, RPT)], sum_out.at[cid, pl.ds(r0, RPT)])


_sc_layer1 = pl.kernel(
    _sc_layer1_body,
    out_type=[jax.ShapeDtypeStruct((2, NPAD, 32), jnp.float32),
              jax.ShapeDtypeStruct((2, NPAD, 16), jnp.float32)],
    mesh=_mesh,
    compiler_params=pltpu.CompilerParams(use_tc_tiling_on_sc=False),
    scratch_types=[
        pltpu.VMEM((NB, BATCH), jnp.int32),
        pltpu.VMEM((NB, BATCH), jnp.int32),
        pltpu.VMEM((BATCH, 32), jnp.float32),
        pltpu.VMEM((BATCH, 32), jnp.float32),
        pltpu.VMEM((BATCH, 16), jnp.float32),
        pltpu.VMEM_SHARED((NPAD, 32), jnp.float32),
        pltpu.VMEM_SHARED((NPAD, 16), jnp.float32),
        pltpu.SemaphoreType.DMA,
        pltpu.SemaphoreType.DMA,
        pltpu.SemaphoreType.DMA,
    ])

_sc_layer2 = pl.kernel(
    _sc_layer2_body,
    out_type=[jax.ShapeDtypeStruct((2, NPAD, 16), jnp.float32)],
    mesh=_mesh,
    compiler_params=pltpu.CompilerParams(use_tc_tiling_on_sc=False),
    scratch_types=[
        pltpu.VMEM((NB, BATCH), jnp.int32),
        pltpu.VMEM((NB, BATCH), jnp.int32),
        pltpu.VMEM((BATCH, 16), jnp.float32),
        pltpu.VMEM((BATCH, 16), jnp.float32),
        pltpu.VMEM_SHARED((NPAD, 16), jnp.float32),
        pltpu.SemaphoreType.DMA,
        pltpu.SemaphoreType.DMA,
    ])


def _tc_in_body(x_ref, wl_ref, wr_ref, bl_ref, p_ref, q_ref):
    x = x_ref[...]
    p_ref[...] = jnp.dot(x, wl_ref[...], preferred_element_type=jnp.float32)
    q_ref[...] = (jnp.dot(x, wr_ref[...], preferred_element_type=jnp.float32)
                  + bl_ref[...])


def _tc_mid_body(sum_ref, cnt_ref, q1_ref, wl_ref, wr_ref, bl_ref,
                 p2_ref, q2_ref):
    s = sum_ref[0, :N, :] + sum_ref[1, :N, :]
    c = cnt_ref[0, :N, :1] + cnt_ref[1, :N, :1]
    h = jnp.maximum(s / jnp.maximum(c, 1.0) + q1_ref[...], 0.0)
    p2_ref[...] = jnp.dot(h, wl_ref[...], preferred_element_type=jnp.float32)
    q2_ref[...] = (jnp.dot(h, wr_ref[...], preferred_element_type=jnp.float32)
                   + bl_ref[...])


def _tc_out_body(sum_ref, cnt_ref, q2_ref, wo_ref, bo_ref, o_ref):
    s = sum_ref[0, :N, :] + sum_ref[1, :N, :]
    c = cnt_ref[0, :N, :1] + cnt_ref[1, :N, :1]
    h = jnp.maximum(s / jnp.maximum(c, 1.0) + q2_ref[...], 0.0)
    o_ref[...] = jax.nn.sigmoid(
        jnp.dot(h, wo_ref[...], preferred_element_type=jnp.float32)
        + bo_ref[...])


_tc_in = pl.pallas_call(
    _tc_in_body,
    out_shape=[jax.ShapeDtypeStruct((N, 32), jnp.float32),
               jax.ShapeDtypeStruct((N, 32), jnp.float32)])
_tc_mid = pl.pallas_call(
    _tc_mid_body,
    out_shape=[jax.ShapeDtypeStruct((N, 16), jnp.float32),
               jax.ShapeDtypeStruct((N, 16), jnp.float32)])
_tc_out = pl.pallas_call(
    _tc_out_body,
    out_shape=jax.ShapeDtypeStruct((N, 8), jnp.float32))


def kernel(x, edge_index, Wl1, bl1, Wr1, Wl2, bl2, Wr2, Wo, bo):
    ei = edge_index.astype(jnp.int32)
    src = jnp.concatenate(
        [ei[0], jnp.zeros((EPAD - E,), jnp.int32)]).reshape(NW, NB, BATCH)
    dst = jnp.concatenate(
        [ei[1], jnp.full((EPAD - E,), N, jnp.int32)]).reshape(NW, NB, BATCH)
    z32 = jnp.zeros((RPT, 32), jnp.float32)
    z16 = jnp.zeros((RPT, 16), jnp.float32)
    ones = jnp.ones((BATCH, 16), jnp.float32)

    p1, q1 = _tc_in(x, Wl1, Wr1, bl1.reshape(1, 32))
    sum1, cnt = _sc_layer1(p1, src, dst, z32, z16, ones)
    p2, q2 = _tc_mid(sum1, cnt, q1, Wl2, Wr2, bl2.reshape(1, 16))
    sum2, = _sc_layer2(p2, src, dst, z16)
    o = _tc_out(sum2, cnt, q2, jnp.tile(Wo, (1, 8)), bo.reshape(1, 1))
    return o[:, :1]
